# Initial kernel scaffold; baseline (speedup 1.0000x reference)
import jax
import jax.numpy as jnp
from jax.experimental import pallas as pl


def kernel(x, rbf, sbf, i, j, idx_kj, idx_ji, params):
    raise NotImplementedError("write your pallas kernel here")


# speedup vs baseline: 1.9061x; 1.9061x over previous
import functools

import jax
import jax.numpy as jnp
from jax import lax
from jax.experimental import pallas as pl
from jax.experimental.pallas import tpu as pltpu
from jax.experimental.pallas import tpu_sc as plsc

F32 = jnp.float32

_B, _N, _E, _T = 2, 10000, 160000, 480000
_H, _R, _NS, _INT, _BAS, _OUT_EMB, _OUT_CH = 64, 6, 7, 32, 8, 96, 1
_NUM_BLOCKS = 2

_BE = 2000
_BT = 3200
_BN = 2000


def _silu(v):
    return v * (1.0 / (1.0 + jnp.exp(-v)))


def _embed_body(rbf_ref, xi_ref, xj_ref, wr_ref, br_ref, wc_ref, uab_ref,
                c_ref, w0_ref, h_ref, y0_ref):
    rbf = rbf_ref[0]
    xi = xi_ref[0]
    xj = xj_ref[0]
    r = _silu(jnp.dot(rbf, wr_ref[...], preferred_element_type=F32, precision=jax.lax.Precision.HIGHEST)
              + br_ref[...])
    t = jnp.dot(r, wc_ref[...], preferred_element_type=F32, precision=jax.lax.Precision.HIGHEST)
    ua = uab_ref[0:1, :]
    ub = uab_ref[1:2, :]
    h = _silu(xi * ua + xj * ub + t + c_ref[...])
    h_ref[0] = h
    g0 = jnp.dot(rbf, w0_ref[...], preferred_element_type=F32, precision=jax.lax.Precision.HIGHEST)
    y0_ref[0] = g0 * h


def _embed_call(rbf, xi, xj, wrT, br, wcT, uab, c, w0T):
    grid = (_B, _E // _BE)
    full = lambda *s: pl.BlockSpec(s, lambda b, e: (0,) * len(s))
    return pl.pallas_call(
        _embed_body,
        grid=grid,
        in_specs=[
            pl.BlockSpec((1, _BE, _R), lambda b, e: (b, e, 0)),
            pl.BlockSpec((1, _BE, 1), lambda b, e: (b, e, 0)),
            pl.BlockSpec((1, _BE, 1), lambda b, e: (b, e, 0)),
            full(_R, _H), full(1, _H), full(_H, _H), full(2, _H), full(1, _H),
            full(_R, _H),
        ],
        out_specs=[
            pl.BlockSpec((1, _BE, _H), lambda b, e: (b, e, 0)),
            pl.BlockSpec((1, _BE, _H), lambda b, e: (b, e, 0)),
        ],
        out_shape=[
            jax.ShapeDtypeStruct((_B, _E, _H), F32),
            jax.ShapeDtypeStruct((_B, _E, _H), F32),
        ],
    )(rbf, xi, xj, wrT, br, wcT, uab, c, w0T)


def _s_body(sbf_ref, s0w_ref, s1w_ref, s0_ref, s1_ref):
    sbf = sbf_ref[0]
    s0_ref[0] = jnp.dot(sbf, s0w_ref[...], preferred_element_type=F32, precision=jax.lax.Precision.HIGHEST)
    s1_ref[0] = jnp.dot(sbf, s1w_ref[...], preferred_element_type=F32, precision=jax.lax.Precision.HIGHEST)


def _s_call(sbf, s0wT, s1wT):
    grid = (_B, _T // _BT)
    full = lambda *s: pl.BlockSpec(s, lambda b, t: (0,) * len(s))
    return pl.pallas_call(
        _s_body,
        grid=grid,
        in_specs=[
            pl.BlockSpec((1, _BT, _NS * _R), lambda b, t: (b, t, 0)),
            full(_NS * _R, _INT), full(_NS * _R, _INT),
        ],
        out_specs=[
            pl.BlockSpec((1, _BT, _INT), lambda b, t: (b, t, 0)),
            pl.BlockSpec((1, _BT, _INT), lambda b, t: (b, t, 0)),
        ],
        out_shape=[
            jax.ShapeDtypeStruct((_B, _T, _INT), F32),
            jax.ShapeDtypeStruct((_B, _T, _INT), F32),
        ],
    )(sbf, s0wT, s1wT)


def _part1_body(h_ref, rbf_ref, jiw_ref, jib_ref, kjw_ref, kjb_ref, r2_ref,
                dw_ref, xji_ref, tab_ref):
    h = h_ref[0]
    rbf = rbf_ref[0]
    xji_ref[0] = _silu(jnp.dot(h, jiw_ref[...], preferred_element_type=F32, precision=jax.lax.Precision.HIGHEST)
                       + jib_ref[...])
    xkj = _silu(jnp.dot(h, kjw_ref[...], preferred_element_type=F32, precision=jax.lax.Precision.HIGHEST)
                + kjb_ref[...])
    xkj = xkj * jnp.dot(rbf, r2_ref[...], preferred_element_type=F32, precision=jax.lax.Precision.HIGHEST)
    tab_ref[0] = _silu(jnp.dot(xkj, dw_ref[...], preferred_element_type=F32, precision=jax.lax.Precision.HIGHEST))


def _part1_call(h, rbf, jiwT, jib, kjwT, kjb, r2T, downT):
    grid = (_B, _E // _BE)
    full = lambda *s: pl.BlockSpec(s, lambda b, e: (0,) * len(s))
    return pl.pallas_call(
        _part1_body,
        grid=grid,
        in_specs=[
            pl.BlockSpec((1, _BE, _H), lambda b, e: (b, e, 0)),
            pl.BlockSpec((1, _BE, _R), lambda b, e: (b, e, 0)),
            full(_H, _H), full(1, _H), full(_H, _H), full(1, _H),
            full(_R, _H), full(_H, _INT),
        ],
        out_specs=[
            pl.BlockSpec((1, _BE, _H), lambda b, e: (b, e, 0)),
            pl.BlockSpec((1, _BE, _INT), lambda b, e: (b, e, 0)),
        ],
        out_shape=[
            jax.ShapeDtypeStruct((_B, _E, _H), F32),
            jax.ShapeDtypeStruct((_B, _E, _INT), F32),
        ],
    )(h, rbf, jiwT, jib, kjwT, kjb, r2T, downT)


def _part2_body(seg_ref, xji_ref, h_ref, rbf_ref, upw_ref, b1w_ref, b1b_ref,
                b2w_ref, b2b_ref, linw_ref, linb_ref, a1w_ref, a1b_ref,
                a2w_ref, a2b_ref, wn_ref, hn_ref, y_ref):
    seg = seg_ref[0]
    xk = _silu(jnp.dot(seg, upw_ref[...], preferred_element_type=F32, precision=jax.lax.Precision.HIGHEST))
    hh = xji_ref[0] + xk
    t = _silu(jnp.dot(hh, b1w_ref[...], preferred_element_type=F32, precision=jax.lax.Precision.HIGHEST)
              + b1b_ref[...])
    hh = hh + _silu(jnp.dot(t, b2w_ref[...], preferred_element_type=F32, precision=jax.lax.Precision.HIGHEST)
                    + b2b_ref[...])
    hh = _silu(jnp.dot(hh, linw_ref[...], preferred_element_type=F32, precision=jax.lax.Precision.HIGHEST)
               + linb_ref[...]) + h_ref[0]
    t = _silu(jnp.dot(hh, a1w_ref[...], preferred_element_type=F32, precision=jax.lax.Precision.HIGHEST)
              + a1b_ref[...])
    hh = hh + _silu(jnp.dot(t, a2w_ref[...], preferred_element_type=F32, precision=jax.lax.Precision.HIGHEST)
                    + a2b_ref[...])
    hn_ref[0] = hh
    g = jnp.dot(rbf_ref[0], wn_ref[...], preferred_element_type=F32, precision=jax.lax.Precision.HIGHEST)
    y_ref[0] = g * hh


def _part2_call(seg, xji, h, rbf, upwT, b1wT, b1b, b2wT, b2b, linwT, linb,
                a1wT, a1b, a2wT, a2b, wnT):
    grid = (_B, _E // _BE)
    full = lambda *s: pl.BlockSpec(s, lambda b, e: (0,) * len(s))
    return pl.pallas_call(
        _part2_body,
        grid=grid,
        in_specs=[
            pl.BlockSpec((1, _BE, _INT), lambda b, e: (b, e, 0)),
            pl.BlockSpec((1, _BE, _H), lambda b, e: (b, e, 0)),
            pl.BlockSpec((1, _BE, _H), lambda b, e: (b, e, 0)),
            pl.BlockSpec((1, _BE, _R), lambda b, e: (b, e, 0)),
            full(_INT, _H), full(_H, _H), full(1, _H), full(_H, _H),
            full(1, _H), full(_H, _H), full(1, _H), full(_H, _H), full(1, _H),
            full(_H, _H), full(1, _H), full(_R, _H),
        ],
        out_specs=[
            pl.BlockSpec((1, _BE, _H), lambda b, e: (b, e, 0)),
            pl.BlockSpec((1, _BE, _H), lambda b, e: (b, e, 0)),
        ],
        out_shape=[
            jax.ShapeDtypeStruct((_B, _E, _H), F32),
            jax.ShapeDtypeStruct((_B, _E, _H), F32),
        ],
    )(seg, xji, h, rbf, upwT, b1wT, b1b, b2wT, b2b, linwT, linb, a1wT, a1b,
      a2wT, a2b, wnT)


def _nodemlp_body(*refs):
    z_refs = refs[:3]
    p_ref = refs[-1]
    acc = jnp.zeros((_BN, _OUT_CH), F32)
    for l in range(3):
        upw, l1w, l1b, l2w, l2b, ow = refs[3 + 6 * l: 3 + 6 * (l + 1)]
        z = z_refs[l][0, 0] + z_refs[l][0, 1]
        y = jnp.dot(z, upw[...], preferred_element_type=F32, precision=jax.lax.Precision.HIGHEST)
        y = _silu(jnp.dot(y, l1w[...], preferred_element_type=F32, precision=jax.lax.Precision.HIGHEST) + l1b[...])
        y = _silu(jnp.dot(y, l2w[...], preferred_element_type=F32, precision=jax.lax.Precision.HIGHEST) + l2b[...])
        acc = acc + jnp.dot(y, ow[...], preferred_element_type=F32, precision=jax.lax.Precision.HIGHEST)
    p_ref[0] = acc


def _nodemlp_call(z_list, wsets):
    grid = (_B, _N // _BN)
    full = lambda *s: pl.BlockSpec(s, lambda b, n: (0,) * len(s))
    in_specs = [pl.BlockSpec((1, 2, _BN, _H), lambda b, n: (b, 0, n, 0))
                for _ in range(3)]
    args = list(z_list)
    for ws in wsets:
        upwT, l1wT, l1b, l2wT, l2b, owT = ws
        in_specs += [full(_H, _OUT_EMB), full(_OUT_EMB, _OUT_EMB),
                     full(1, _OUT_EMB), full(_OUT_EMB, _OUT_EMB),
                     full(1, _OUT_EMB), full(_OUT_EMB, _OUT_CH)]
        args += [upwT, l1wT, l1b, l2wT, l2b, owT]
    return pl.pallas_call(
        _nodemlp_body,
        grid=grid,
        in_specs=in_specs,
        out_specs=pl.BlockSpec((1, _BN, _OUT_CH), lambda b, n: (b, n, 0)),
        out_shape=jax.ShapeDtypeStruct((_B, _N, _OUT_CH), F32),
    )(*args)


_NC, _NSUB, _L = 2, 16, 16
_NW = _NC * _NSUB
_NRANGE = 10
_E4 = _E // _NRANGE
_TPAD = 524288
_TSL = _TPAD // _NSUB
_CAP = _TSL + 256
_CCH = 2048
_NQ = _CCH // 128
_TSTRIPE = _E4 // _NSUB
_NZ = 632
_NROWS = _NZ * _NSUB
_GXP = _E // _NW
_EPT = _E // _NW
_ECH = 128
_ECHN = _EPT // _ECH


@functools.cache
def _mesh():
    return plsc.VectorSubcoreMesh(core_axis_name="c", subcore_axis_name="s",
                                  num_cores=_NC, num_subcores=_NSUB)


def _gx_body(x0_hbm, x1_hbm, i_hbm, j_hbm, oi0, oi1, oj0, oj1,
             idx_v, res_v, sem):
    c = lax.axis_index("c")
    s = lax.axis_index("s")
    wid = s * _NC + c
    base = wid * _GXP
    outs = ((oi0, oi1), (oj0, oj1))
    x_refs = (x0_hbm, x1_hbm)
    nfull = _GXP // 128
    for sel in range(2):
        idx_hbm = i_hbm if sel == 0 else j_hbm
        pltpu.sync_copy(idx_hbm.at[pl.ds(base, _GXP)], idx_v)
        for b in range(_B):
            cps = []
            for k in range(nfull):
                cps.append(pltpu.async_copy(
                    x_refs[b].at[idx_v.at[pl.ds(k * 128, 128)]],
                    res_v.at[pl.ds(k * 128, 128)], sem))
            cps.append(pltpu.async_copy(
                x_refs[b].at[idx_v.at[pl.ds(nfull * 128, 8)]],
                res_v.at[pl.ds(nfull * 128, 8)], sem))
            for cp in cps:
                cp.wait()
            pltpu.sync_copy(res_v, outs[sel][b].at[pl.ds(base, _GXP)])


def _gx_call(x0, x1, i, j):
    f = pl.kernel(
        _gx_body,
        out_type=[jax.ShapeDtypeStruct((_E,), F32) for _ in range(4)],
        mesh=_mesh(),
        scratch_types=[
            pltpu.VMEM((_GXP,), jnp.int32),
            pltpu.VMEM((_GXP,), F32),
            pltpu.SemaphoreType.DMA,
        ],
    )
    return f(x0, x1, i, j)


def _compact_body(kj_hbm, ji_hbm, tl_hbm, kl_hbm, jl_hbm, cnt_hbm,
                  kjst, jist, sbuf, tvb, kvb, jvb, posq, padv, cntv, sem):
    c = lax.axis_index("c")
    s = lax.axis_index("s")
    sbase = s * _TSL
    nch = _TSL // _CCH
    for rl in range(_NRANGE // _NC):
        r = (_NRANGE // _NC) * c + rl
        lo = r * _E4
        lbase = (r * _NSUB + s) * _CAP
        sbuf[pl.ds(0, _L)] = jnp.zeros((_L,), jnp.int32)

        def chbody(ch, cnt):
            pltpu.sync_copy(kj_hbm.at[pl.ds(sbase + ch * _CCH, _CCH)], kjst)
            pltpu.sync_copy(ji_hbm.at[pl.ds(sbase + ch * _CCH, _CCH)], jist)
            tb = sbase + ch * _CCH
            cps = []
            for q in range(_NQ):
                pq = posq[q]

                def vbody(k, cnt, q=q):
                    kk = q * 8 + k
                    ji = jist[pl.ds(kk * _L, _L)]
                    kj = kjst[pl.ds(kk * _L, _L)]
                    iota = lax.iota(jnp.int32, _L)
                    tv = jnp.full((_L,), tb + kk * _L, jnp.int32) + iota
                    lo_v = jnp.full((_L,), lo, jnp.int32)
                    m = (ji >= lo_v) & (ji < lo_v + _E4)
                    mi = jnp.where(m, jnp.full((_L,), 1, jnp.int32),
                                   jnp.full((_L,), 0, jnp.int32))
                    sbuf[pl.ds(_L, _L)] = mi
                    v = mi + sbuf[pl.ds(_L - 1, _L)]
                    sbuf[pl.ds(_L, _L)] = v
                    v = v + sbuf[pl.ds(_L - 2, _L)]
                    sbuf[pl.ds(_L, _L)] = v
                    v = v + sbuf[pl.ds(_L - 4, _L)]
                    sbuf[pl.ds(_L, _L)] = v
                    v = v + sbuf[pl.ds(_L - 8, _L)]
                    total = v[_L - 1]
                    base_v = jnp.full((_L,), lbase + cnt - 1, jnp.int32)
                    trash_v = jnp.bitwise_and(
                        jnp.full((_L,), kk * _L, jnp.int32) + iota,
                        jnp.full((_L,), 127, jnp.int32))
                    trash_v = trash_v + jnp.full(
                        (_L,), lbase + _CAP - 128, jnp.int32)
                    pos = jnp.where(m, base_v + v, trash_v)
                    posq[q][pl.ds(k * _L, _L)] = pos
                    off = pl.ds(kk * _L, _L)
                    tvb[off] = tv
                    kvb[off] = kj
                    jvb[off] = ji - lo_v
                    return cnt + total

                cnt = lax.fori_loop(0, 8, vbody, cnt)
                src_off = pl.ds(q * 128, 128)
                cps.append(pltpu.async_copy(
                    tvb.at[src_off], tl_hbm.at[pq], sem))
                cps.append(pltpu.async_copy(
                    kvb.at[src_off], kl_hbm.at[pq], sem))
                cps.append(pltpu.async_copy(
                    jvb.at[src_off], jl_hbm.at[pq], sem))
            for cp in cps:
                cp.wait()
            return cnt

        cnt0 = lax.fori_loop(0, nch, chbody, jnp.int32(0))

        def padfill(q, _):
            padv[pl.ds(q * _L, _L)] = (
                jnp.full((_L,), lbase + cnt0 + q * _L, jnp.int32)
                + lax.iota(jnp.int32, _L))
            off = pl.ds(q * _L, _L)
            tvb[off] = jnp.zeros((_L,), jnp.int32)
            kvb[off] = jnp.zeros((_L,), jnp.int32)
            jvb[off] = jnp.full((_L,), _E4, jnp.int32)
            return 0

        lax.fori_loop(0, 8, padfill, 0)
        src_off = pl.ds(0, 128)
        pltpu.async_copy(tvb.at[src_off], tl_hbm.at[padv], sem).wait()
        pltpu.async_copy(kvb.at[src_off], kl_hbm.at[padv], sem).wait()
        pltpu.async_copy(jvb.at[src_off], jl_hbm.at[padv], sem).wait()
        cntv[pl.ds(0, _L)] = jnp.full((_L,), cnt0, jnp.int32)
        pltpu.sync_copy(cntv.at[pl.ds(0, 8)],
                        cnt_hbm.at[pl.ds((r * _NSUB + s) * 8, 8)])


def _compact_call(idx_kj, idx_ji):
    f = pl.kernel(
        _compact_body,
        out_type=[
            jax.ShapeDtypeStruct((_NRANGE * _NSUB * _CAP,), jnp.int32),
            jax.ShapeDtypeStruct((_NRANGE * _NSUB * _CAP,), jnp.int32),
            jax.ShapeDtypeStruct((_NRANGE * _NSUB * _CAP,), jnp.int32),
            jax.ShapeDtypeStruct((_NRANGE * _NSUB * 8,), jnp.int32),
        ],
        mesh=_mesh(),
        scratch_types=[
            pltpu.VMEM((_CCH,), jnp.int32),
            pltpu.VMEM((_CCH,), jnp.int32),
            pltpu.VMEM((2 * _L,), jnp.int32),
            pltpu.VMEM((_CCH,), jnp.int32),
            pltpu.VMEM((_CCH,), jnp.int32),
            pltpu.VMEM((_CCH,), jnp.int32),
            [pltpu.VMEM((128,), jnp.int32) for _ in range(_NQ)],
            pltpu.VMEM((128,), jnp.int32),
            pltpu.VMEM((_L,), jnp.int32),
            pltpu.SemaphoreType.DMA,
        ],
    )
    return f(idx_kj, idx_ji)


def _triplet_body(s0_hbm, s1_hbm, tab0_hbm, tab1_hbm, tl_hbm, kl_hbm, jl_hbm,
                  cnt_hbm, z_hbm, seg0_hbm, seg1_hbm,
                  tch, kch, jch, svals, tvals, cntv, acc, sem1, sem2):
    c = lax.axis_index("c")
    s = lax.axis_index("s")
    s_refs = (s0_hbm, s1_hbm)
    tab_refs = (tab0_hbm, tab1_hbm)
    seg_refs = (seg0_hbm, seg1_hbm)
    for rl in range(_NRANGE // _NC):
        r = (_NRANGE // _NC) * c + rl
        lbase = (r * _NSUB + s) * _CAP
        for b in range(_B):
            pltpu.sync_copy(z_hbm, acc.at[pl.ds(s * _TSTRIPE, _TSTRIPE)])
            plsc.subcore_barrier()
            pltpu.sync_copy(cnt_hbm.at[pl.ds((r * _NSUB + s) * 8, 8)],
                            cntv.at[pl.ds(0, 8)])
            cnt = cntv[pl.ds(0, _L)][0]

            def chunk(ci, _):
                @pl.when(ci * 128 < cnt)
                def _():
                    off = lbase + ci * 128
                    pltpu.sync_copy(tl_hbm.at[pl.ds(off, 128)], tch)
                    pltpu.sync_copy(kl_hbm.at[pl.ds(off, 128)], kch)
                    pltpu.sync_copy(jl_hbm.at[pl.ds(off, 128)], jch)
                    cp1 = pltpu.async_copy(s_refs[b].at[tch], svals, sem1)
                    cp2 = pltpu.async_copy(tab_refs[b].at[kch], tvals, sem2)
                    cp1.wait()
                    cp2.wait()
                    for rw in range(128):
                        for hf in (0, _L):
                            svals[rw, pl.ds(hf, _L)] = (
                                svals[rw, pl.ds(hf, _L)]
                                * tvals[rw, pl.ds(hf, _L)])
                    pltpu.sync_copy(svals, acc.at[jch], add=True)
                return 0

            lax.fori_loop(0, _CAP // 128, chunk, 0)
            plsc.subcore_barrier()
            pltpu.sync_copy(
                acc.at[pl.ds(s * _TSTRIPE, _TSTRIPE)],
                seg_refs[b].at[pl.ds(r * _E4 + s * _TSTRIPE, _TSTRIPE)])
            plsc.subcore_barrier()


def _triplet_call(s0, s1, tab0, tab1, tl, kl, jl, cnts, zeros_t):
    f = pl.kernel(
        _triplet_body,
        out_type=[jax.ShapeDtypeStruct((_E, _INT), F32),
                  jax.ShapeDtypeStruct((_E, _INT), F32)],
        mesh=_mesh(),
        scratch_types=[
            pltpu.VMEM((128,), jnp.int32),
            pltpu.VMEM((128,), jnp.int32),
            pltpu.VMEM((128,), jnp.int32),
            pltpu.VMEM((128, _INT), F32),
            pltpu.VMEM((128, _INT), F32),
            pltpu.VMEM((_L,), jnp.int32),
            pltpu.VMEM_SHARED((_E4 + 8, _INT), F32),
            pltpu.SemaphoreType.DMA,
            pltpu.SemaphoreType.DMA,
        ],
        compiler_params=pltpu.CompilerParams(use_tc_tiling_on_sc=False),
    )
    return f(s0, s1, tab0, tab1, tl, kl, jl, cnts, zeros_t)


def _node_body(y0_hbm, y1_hbm, i_hbm, z_hbm, o0_hbm, o1_hbm,
               ich, ich8, ybuf, ybuf8, acc):
    c = lax.axis_index("c")
    s = lax.axis_index("s")
    wid = c * _NSUB + s
    base_e = c * (_E // _NC) + s * _EPT
    y_refs = (y0_hbm, y1_hbm)
    o_refs = (o0_hbm, o1_hbm)
    for b in range(_B):
        pltpu.sync_copy(z_hbm, acc.at[pl.ds(s * _NZ, _NZ)])
        plsc.subcore_barrier()

        def chunk(q, _):
            off = base_e + q * _ECH
            pltpu.sync_copy(i_hbm.at[pl.ds(off, _ECH)], ich)
            pltpu.sync_copy(y_refs[b].at[pl.ds(off, _ECH)], ybuf)
            pltpu.sync_copy(ybuf, acc.at[ich], add=True)
            return 0

        lax.fori_loop(0, _ECHN, chunk, 0)
        off8 = base_e + _ECHN * _ECH
        pltpu.sync_copy(i_hbm.at[pl.ds(off8, 8)], ich8)
        pltpu.sync_copy(y_refs[b].at[pl.ds(off8, 8)], ybuf8)
        pltpu.sync_copy(ybuf8, acc.at[ich8], add=True)
        plsc.subcore_barrier()
        pltpu.sync_copy(acc.at[pl.ds(s * _NZ, _NZ)],
                        o_refs[b].at[c, pl.ds(s * _NZ, _NZ)])
        plsc.subcore_barrier()


def _node_call(y0, y1, i, zeros_n):
    f = pl.kernel(
        _node_body,
        out_type=[jax.ShapeDtypeStruct((_NC, _NROWS, _H), F32),
                  jax.ShapeDtypeStruct((_NC, _NROWS, _H), F32)],
        mesh=_mesh(),
        scratch_types=[
            pltpu.VMEM((_ECH,), jnp.int32),
            pltpu.VMEM((8,), jnp.int32),
            pltpu.VMEM((_ECH, _H), F32),
            pltpu.VMEM((8, _H), F32),
            pltpu.VMEM_SHARED((_NROWS, _H), F32),
        ],
        compiler_params=pltpu.CompilerParams(use_tc_tiling_on_sc=False),
    )
    return f(y0, y1, i, zeros_n)


def kernel(x, rbf, sbf, i, j, idx_kj, idx_ji, params):
    emb = params['emb']
    w = emb['emb_W'][:, 0]
    be = emb['emb_b']
    lin_W = emb['lin_W']
    A, Bm, C = lin_W[:, :_H], lin_W[:, _H:2 * _H], lin_W[:, 2 * _H:]
    uab = jnp.stack([A @ w, Bm @ w])
    cvec = (A @ be + Bm @ be + emb['lin_b'])[None, :]

    xi0, xi1, xj0, xj1 = _gx_call(x[0, :, 0], x[1, :, 0], i, j)
    xi = jnp.stack([xi0, xi1])
    xj = jnp.stack([xj0, xj1])
    pad_n = _TPAD - _T
    kj_p = jnp.concatenate([idx_kj, jnp.zeros((pad_n,), jnp.int32)])
    ji_p = jnp.concatenate([idx_ji,
                            jnp.full((pad_n,), 1 << 29, jnp.int32)])
    tl, kl, jl, cnts = _compact_call(kj_p, ji_p)
    zeros_t = jnp.zeros((_TSTRIPE, _INT), F32)
    zeros_n = jnp.zeros((_NZ, _H), F32)

    def node_call(y):
        o0, o1 = _node_call(y[0], y[1], i, zeros_n)
        return jnp.stack([o0, o1])

    h, y0 = _embed_call(
        rbf, xi[..., None], xj[..., None], emb['lin_rbf_W'].T,
        emb['lin_rbf_b'][None, :], C.T, uab, cvec,
        params['out'][0]['rbf_W'].T)

    z_list = [node_call(y0)]
    for b in range(_NUM_BLOCKS):
        p = params['inter'][b]
        if b == 0:
            s0, s1 = _s_call(sbf, (p['sbf2'] @ p['sbf1']).T,
                             (params['inter'][1]['sbf2']
                              @ params['inter'][1]['sbf1']).T)
        s = s0 if b == 0 else s1
        xji, table = _part1_call(
            h, rbf, p['ji_W'].T, p['ji_b'][None, :], p['kj_W'].T,
            p['kj_b'][None, :], (p['rbf2'] @ p['rbf1']).T, p['down'].T)
        sg0, sg1 = _triplet_call(s[0], s[1], table[0], table[1], tl, kl, jl,
                                 cnts, zeros_t)
        seg = jnp.stack([sg0, sg1])
        bl = p['before'][0]
        al = p['after'][0]
        h, y = _part2_call(
            seg, xji, h, rbf, p['up'].T, bl['l1_W'].T, bl['l1_b'][None, :],
            bl['l2_W'].T, bl['l2_b'][None, :], p['lin_W'].T,
            p['lin_b'][None, :], al['l1_W'].T, al['l1_b'][None, :],
            al['l2_W'].T, al['l2_b'][None, :],
            params['out'][b + 1]['rbf_W'].T)
        z_list.append(node_call(y))

    wsets = []
    for l in range(3):
        po = params['out'][l]
        wsets.append((po['up_W'].T, po['lins'][0]['W'].T,
                      po['lins'][0]['b'][None, :], po['lins'][1]['W'].T,
                      po['lins'][1]['b'][None, :], po['out_W'].T))
    return _nodemlp_call(z_list, wsets)
